# baseline (speedup 1.0000x reference)
import jax
import jax.numpy as jnp
from jax.experimental import pallas as pl
from jax.experimental.pallas import tpu as pltpu

_CAP_ROWS = 262144
_ROW_DIM = 128
_SLOT_ROWS = 32768
_NSLOT = 3

_SIZES = [2048, 2048, 4096, 8192, 16384] + [32768] * 6 + [16384, 8192, 4096, 2048, 2048]
_OFFS = []
_o = 0
for _s in _SIZES:
    _OFFS.append(_o)
    _o += _s
assert _o == _CAP_ROWS
_NCH = len(_SIZES)


def _pushback_body(end_ref, data_ref, buf_ref, out_ref, slots, rsems, wsems):
    end = end_ref[0]

    def rd(k):
        slot = k % _NSLOT
        return pltpu.make_async_copy(
            buf_ref.at[pl.ds(_OFFS[k], _SIZES[k]), :],
            slots.at[slot, pl.ds(0, _SIZES[k]), :],
            rsems.at[slot],
        )

    def wr(k):
        slot = k % _NSLOT
        return pltpu.make_async_copy(
            slots.at[slot, pl.ds(0, _SIZES[k]), :],
            out_ref.at[pl.ds(_OFFS[k], _SIZES[k]), :],
            wsems.at[slot],
        )

    for k in range(_NSLOT):
        rd(k).start()
    for k in range(_NCH):
        nxt = k + 1
        if nxt < _NCH and nxt >= _NSLOT:
            wr(nxt - _NSLOT).wait()
            rd(nxt).start()
        rd(k).wait()

        local = end - _OFFS[k]

        @pl.when((local >= 0) & (local < _SIZES[k]))
        def _():
            slots[k % _NSLOT, pl.ds(local, 1), :] = data_ref[...]

        wr(k).start()
    for k in range(_NCH - _NSLOT, _NCH):
        wr(k).wait()


def kernel(data, buffer, start_included, end_excluded, length):
    end = jnp.asarray(end_excluded, jnp.int32).reshape(1)
    data2 = data.reshape(1, _ROW_DIM)
    return pl.pallas_call(
        _pushback_body,
        in_specs=[
            pl.BlockSpec(memory_space=pltpu.SMEM),
            pl.BlockSpec(memory_space=pltpu.VMEM),
            pl.BlockSpec(memory_space=pl.ANY),
        ],
        out_specs=pl.BlockSpec(memory_space=pl.ANY),
        out_shape=jax.ShapeDtypeStruct((_CAP_ROWS, _ROW_DIM), jnp.float32),
        scratch_shapes=[
            pltpu.VMEM((_NSLOT, _SLOT_ROWS, _ROW_DIM), jnp.float32),
            pltpu.SemaphoreType.DMA((_NSLOT,)),
            pltpu.SemaphoreType.DMA((_NSLOT,)),
        ],
    )(end, data2, buffer)


# speedup vs baseline: 1.0224x; 1.0224x over previous
import jax
import jax.numpy as jnp
from jax.experimental import pallas as pl
from jax.experimental.pallas import tpu as pltpu

_CAP_ROWS = 262144
_ROW_DIM = 128
_BLOCK = 16384


def _pushback_body(end_ref, data_ref, buf_ref, out_ref):
    out_ref[...] = buf_ref[...]
    i = pl.program_id(0)
    local = end_ref[0] - i * _BLOCK

    @pl.when((local >= 0) & (local < _BLOCK))
    def _():
        out_ref[pl.ds(local, 1), :] = data_ref[...]


def kernel(data, buffer, start_included, end_excluded, length):
    end = jnp.asarray(end_excluded, jnp.int32).reshape(1)
    data2 = data.reshape(1, _ROW_DIM)
    return pl.pallas_call(
        _pushback_body,
        grid=(_CAP_ROWS // _BLOCK,),
        in_specs=[
            pl.BlockSpec(memory_space=pltpu.SMEM),
            pl.BlockSpec((1, _ROW_DIM), lambda i: (0, 0)),
            pl.BlockSpec((_BLOCK, _ROW_DIM), lambda i: (i, 0)),
        ],
        out_specs=pl.BlockSpec((_BLOCK, _ROW_DIM), lambda i: (i, 0)),
        out_shape=jax.ShapeDtypeStruct((_CAP_ROWS, _ROW_DIM), jnp.float32),
        compiler_params=pltpu.CompilerParams(
            dimension_semantics=("arbitrary",),
        ),
    )(end, data2, buffer)
